# Initial kernel scaffold; baseline (speedup 1.0000x reference)
import jax
import jax.numpy as jnp
from jax.experimental import pallas as pl


def kernel(out_preds, out_targets, tl, tv, x_rep, in_x, in_l, in_v, in_n):
    raise NotImplementedError("write your pallas kernel here")


# speedup vs baseline: 6.2443x; 6.2443x over previous
import numpy as np
import jax
import jax.numpy as jnp
from jax.experimental import pallas as pl
from jax.experimental.pallas import tpu as pltpu

_B, _T, _C, _H, _W = 4, 4, 8, 224, 224
_R = _B * _T
_N = _H * _W
_K = _N // 2
_S = 8
_L = _N // _S
_DENOM = float(_R * _K * _C)

_G = np.asarray(
    jax.random.gumbel(jax.random.key(42), (_R, _N), dtype=jnp.float32)
).reshape(_R, _S, _L)

_INT_MIN = np.int32(-2147483648)


def _mae_body(t_ref, p_ref, g_ref, o_ref):
    r = pl.program_id(0)
    t = t_ref[0]
    p = p_ref[0]
    g = g_ref[0]

    norm = jnp.sqrt(jnp.sum(t * t, axis=0)) + 0.5
    score = jnp.log(norm) + g
    d = jnp.sum(jnp.abs(p - t), axis=0)

    u = jax.lax.bitcast_convert_type(score, jnp.int32)
    key = jnp.where(u >= 0, u, _INT_MIN - u)

    kmin = jnp.min(key)
    kmax = jnp.max(key)

    def body(_, lohi):
        lo, hi = lohi
        mid = (lo >> 1) + (hi >> 1) + (lo & hi & 1)
        cnt = jnp.sum((key >= mid).astype(jnp.int32))
        pred = cnt >= _K
        return jnp.where(pred, mid, lo), jnp.where(pred, hi, mid)

    lo, _ = jax.lax.fori_loop(0, 32, body, (kmin, kmax + 1))
    tau = lo

    mask_gt = key > tau
    mask_eq = key == tau
    count_gt = jnp.sum(mask_gt.astype(jnp.float32))
    count_eq = jnp.sum(mask_eq.astype(jnp.float32))
    sum_gt = jnp.sum(jnp.where(mask_gt, d, 0.0))
    sum_eq = jnp.sum(jnp.where(mask_eq, d, 0.0))
    need = jnp.float32(_K) - count_gt
    row_sum = sum_gt + need * sum_eq / count_eq

    @pl.when(r == 0)
    def _():
        o_ref[0, 0] = 0.0

    acc = o_ref[0, 0] + row_sum

    @pl.when(r == _R - 1)
    def _():
        o_ref[0, 0] = acc * (1.0 / _DENOM)

    @pl.when(r != _R - 1)
    def _():
        o_ref[0, 0] = acc


def kernel(out_preds, out_targets, tl, tv, x_rep, in_x, in_l, in_v, in_n):
    t = out_targets.reshape(_R, _C, _S, _L)
    p = out_preds.reshape(_R, _C, _S, _L)
    g = jnp.asarray(_G)
    out = pl.pallas_call(
        _mae_body,
        grid=(_R,),
        in_specs=[
            pl.BlockSpec((1, _C, _S, _L), lambda r: (r, 0, 0, 0)),
            pl.BlockSpec((1, _C, _S, _L), lambda r: (r, 0, 0, 0)),
            pl.BlockSpec((1, _S, _L), lambda r: (r, 0, 0)),
        ],
        out_specs=pl.BlockSpec((1, 1), lambda r: (0, 0), memory_space=pltpu.SMEM),
        out_shape=jax.ShapeDtypeStruct((1, 1), jnp.float32),
    )(t, p, g)
    return out[0, 0]


# speedup vs baseline: 9.6967x; 1.5529x over previous
import numpy as np
import jax
import jax.numpy as jnp
from jax.experimental import pallas as pl
from jax.experimental.pallas import tpu as pltpu

_B, _T, _C, _H, _W = 4, 4, 8, 224, 224
_R = _B * _T
_N = _H * _W
_K = _N // 2
_S = 8
_L = _N // _S
_DENOM = float(_R * _K * _C)

_G = np.asarray(
    jax.random.gumbel(jax.random.key(42), (_R, _N), dtype=jnp.float32)
).reshape(_R, _S, _L)

_INT_MIN = np.int32(-2147483648)


def _mae_body(t_ref, p_ref, g_ref, o_ref, key_ref, d_ref):
    r = pl.program_id(0)
    t = t_ref[0]
    p = p_ref[0]
    g = g_ref[0]

    norm = jnp.sqrt(jnp.sum(t * t, axis=0)) + 0.5
    score = jnp.log(norm) + g
    d = jnp.sum(jnp.abs(p - t), axis=0)

    u = jax.lax.bitcast_convert_type(score, jnp.int32)
    key_ref[r] = jnp.where(u >= 0, u, _INT_MIN - u)
    d_ref[r] = d

    @pl.when(r == _R - 1)
    def _phase_b():
        key = key_ref[...]
        dd = d_ref[...]
        kmin = jnp.min(key, axis=(1, 2), keepdims=True)
        kmax = jnp.max(key, axis=(1, 2), keepdims=True)

        def body(_, lohi):
            lo, hi = lohi
            mid = (lo >> 1) + (hi >> 1) + (lo & hi & 1)
            cnt = jnp.sum((key >= mid).astype(jnp.int32), axis=(1, 2),
                          keepdims=True)
            pred = cnt >= _K
            return jnp.where(pred, mid, lo), jnp.where(pred, hi, mid)

        lo, _ = jax.lax.fori_loop(0, 32, body, (kmin, kmax + 1))
        tau = lo

        mask_gt = key > tau
        mask_eq = key == tau
        count_gt = jnp.sum(mask_gt.astype(jnp.float32), axis=(1, 2),
                           keepdims=True)
        count_eq = jnp.sum(mask_eq.astype(jnp.float32), axis=(1, 2),
                           keepdims=True)
        sum_gt = jnp.sum(jnp.where(mask_gt, dd, 0.0), axis=(1, 2),
                         keepdims=True)
        sum_eq = jnp.sum(jnp.where(mask_eq, dd, 0.0), axis=(1, 2),
                         keepdims=True)
        need = jnp.float32(_K) - count_gt
        total = jnp.sum(sum_gt + need * sum_eq / count_eq)
        o_ref[0, 0] = total * (1.0 / _DENOM)


def kernel(out_preds, out_targets, tl, tv, x_rep, in_x, in_l, in_v, in_n):
    t = out_targets.reshape(_R, _C, _S, _L)
    p = out_preds.reshape(_R, _C, _S, _L)
    g = jnp.asarray(_G)
    out = pl.pallas_call(
        _mae_body,
        grid=(_R,),
        in_specs=[
            pl.BlockSpec((1, _C, _S, _L), lambda r: (r, 0, 0, 0)),
            pl.BlockSpec((1, _C, _S, _L), lambda r: (r, 0, 0, 0)),
            pl.BlockSpec((1, _S, _L), lambda r: (r, 0, 0)),
        ],
        out_specs=pl.BlockSpec((1, 1), lambda r: (0, 0), memory_space=pltpu.SMEM),
        out_shape=jax.ShapeDtypeStruct((1, 1), jnp.float32),
        scratch_shapes=[
            pltpu.VMEM((_R, _S, _L), jnp.int32),
            pltpu.VMEM((_R, _S, _L), jnp.float32),
        ],
    )(t, p, g)
    return out[0, 0]


# speedup vs baseline: 23.0750x; 2.3797x over previous
import numpy as np
import jax
import jax.numpy as jnp
from jax.experimental import pallas as pl
from jax.experimental.pallas import tpu as pltpu

_B, _T, _C, _H, _W = 4, 4, 8, 224, 224
_R = _B * _T
_N = _H * _W
_K = _N // 2
_DENOM = float(_R * _K * _C)


def _np_threefry2x32(k0, k1, x0, x1):
    def rotl(x, d):
        return ((x << np.uint32(d)) | (x >> np.uint32(32 - d))).astype(np.uint32)

    ks0, ks1 = np.uint32(k0), np.uint32(k1)
    ks2 = np.uint32(ks0 ^ ks1 ^ np.uint32(0x1BD11BDA))
    ks = [ks0, ks1, ks2]
    rotations = [(13, 15, 26, 6), (17, 29, 16, 24)]
    x0 = (x0 + ks0).astype(np.uint32)
    x1 = (x1 + ks1).astype(np.uint32)
    for i in range(5):
        for r in rotations[i % 2]:
            x0 = (x0 + x1).astype(np.uint32)
            x1 = rotl(x1, r)
            x1 = (x1 ^ x0).astype(np.uint32)
        x0 = (x0 + ks[(i + 1) % 3]).astype(np.uint32)
        x1 = (x1 + ks[(i + 2) % 3] + np.uint32(i + 1)).astype(np.uint32)
    return x0, x1


def _np_uniform_table(seed, size):
    k0 = np.uint32(np.uint64(seed) >> np.uint64(32))
    k1 = np.uint32(np.uint64(seed) & np.uint64(0xFFFFFFFF))
    lo = np.arange(size, dtype=np.uint32)
    hi = np.zeros(size, dtype=np.uint32)
    o0, o1 = _np_threefry2x32(k0, k1, hi, lo)
    bits = o0 ^ o1
    fb = (bits >> np.uint32(9)) | np.uint32(0x3F800000)
    floats = fb.view(np.float32) - np.float32(1.0)
    tiny = np.float32(np.finfo(np.float32).tiny)
    return np.maximum(tiny, floats * (np.float32(1.0) - tiny) + tiny)


_U = _np_uniform_table(42, _R * _N).reshape(_R, _H, _W)

_INT_MIN = np.int32(-2147483648)


def _mae_body(t_ref, p_ref, g_ref, o_ref, key_ref, d_ref):
    r = pl.program_id(0)
    t = t_ref[0]
    p = p_ref[0]
    g = g_ref[0]

    norm = jnp.sqrt(jnp.sum(t * t, axis=0)) + 0.5
    gumb = -jnp.log(-jnp.log(g))
    score = jnp.log(norm) + gumb
    d = jnp.sum(jnp.abs(p - t), axis=0)

    u = jax.lax.bitcast_convert_type(score, jnp.int32)
    key_ref[r] = jnp.where(u >= 0, u, _INT_MIN - u)
    d_ref[r] = d

    @pl.when(r == _R - 1)
    def _phase_b():
        key = key_ref[...]
        dd = d_ref[...]
        kmin = jnp.min(key, axis=(1, 2), keepdims=True)
        kmax = jnp.max(key, axis=(1, 2), keepdims=True)

        def body(_, lohi):
            lo, hi = lohi
            mid = (lo >> 1) + (hi >> 1) + (lo & hi & 1)
            cnt = jnp.sum((key >= mid).astype(jnp.int32), axis=(1, 2),
                          keepdims=True)
            pred = cnt >= _K
            return jnp.where(pred, mid, lo), jnp.where(pred, hi, mid)

        lo, _ = jax.lax.fori_loop(0, 32, body, (kmin, kmax + 1))
        tau = lo

        mask_gt = key > tau
        mask_eq = key == tau
        count_gt = jnp.sum(mask_gt.astype(jnp.float32), axis=(1, 2),
                           keepdims=True)
        count_eq = jnp.sum(mask_eq.astype(jnp.float32), axis=(1, 2),
                           keepdims=True)
        sum_gt = jnp.sum(jnp.where(mask_gt, dd, 0.0), axis=(1, 2),
                         keepdims=True)
        sum_eq = jnp.sum(jnp.where(mask_eq, dd, 0.0), axis=(1, 2),
                         keepdims=True)
        need = jnp.float32(_K) - count_gt
        total = jnp.sum(sum_gt + need * sum_eq / count_eq)
        o_ref[0, 0] = total * (1.0 / _DENOM)


def kernel(out_preds, out_targets, tl, tv, x_rep, in_x, in_l, in_v, in_n):
    t = out_targets.reshape(_R, _C, _H, _W)
    p = out_preds.reshape(_R, _C, _H, _W)
    g = jnp.asarray(_U)
    out = pl.pallas_call(
        _mae_body,
        grid=(_R,),
        in_specs=[
            pl.BlockSpec((1, _C, _H, _W), lambda r: (r, 0, 0, 0)),
            pl.BlockSpec((1, _C, _H, _W), lambda r: (r, 0, 0, 0)),
            pl.BlockSpec((1, _H, _W), lambda r: (r, 0, 0)),
        ],
        out_specs=pl.BlockSpec((1, 1), lambda r: (0, 0), memory_space=pltpu.SMEM),
        out_shape=jax.ShapeDtypeStruct((1, 1), jnp.float32),
        scratch_shapes=[
            pltpu.VMEM((_R, _H, _W), jnp.int32),
            pltpu.VMEM((_R, _H, _W), jnp.float32),
        ],
    )(t, p, g)
    return out[0, 0]
